# Initial kernel scaffold; baseline (speedup 1.0000x reference)
import jax
import jax.numpy as jnp
from jax.experimental import pallas as pl


def kernel(x0, y0, rays):
    raise NotImplementedError("write your pallas kernel here")


# speedup vs baseline: 71.9792x; 71.9792x over previous
import functools

import jax
import jax.numpy as jnp
from jax import lax
from jax.experimental import pallas as pl
from jax.experimental.pallas import tpu as pltpu
from jax.experimental.pallas import tpu_sc as plsc

IMG_SIZE = 512
NUM_CORES = 2
NUM_SUBCORES = 16
NUM_WORKERS = NUM_CORES * NUM_SUBCORES
LANES = 16
CHUNK = 4096
IDX_PER_STREAM = 128
GROUPS = CHUNK // LANES


def _build_patch_table(rays):
    r = rays
    rx = jnp.concatenate([r[:, 1:], r[:, -1:]], axis=1)
    ry = jnp.concatenate([r[1:], r[-1:]], axis=0)
    rxy = jnp.concatenate([rx[1:], rx[-1:]], axis=0)
    pad = jnp.zeros((IMG_SIZE, IMG_SIZE, 4), jnp.float32)
    return jnp.concatenate([r, rx, ry, rxy, pad], axis=-1).reshape(
        IMG_SIZE * IMG_SIZE, 16)


def _sc_body(n_per_worker, x0_hbm, y0_hbm, patch_hbm, out_hbm,
             xv, yv, idxv, rows, outv, sem):
    wid = lax.axis_index("s") * NUM_CORES + lax.axis_index("c")
    base0 = wid * n_per_worker
    n_chunks = n_per_worker // CHUNK

    @pl.loop(0, n_chunks)
    def _chunk(c):
        base = base0 + c * CHUNK
        pltpu.sync_copy(x0_hbm.at[pl.ds(base, CHUNK)], xv)
        pltpu.sync_copy(y0_hbm.at[pl.ds(base, CHUNK)], yv)

        @pl.loop(0, GROUPS)
        def _index(g):
            q = g * LANES
            xq = xv[pl.ds(q, LANES)]
            yq = yv[pl.ds(q, LANES)]
            x1i = jnp.clip(xq.astype(jnp.int32), 0, IMG_SIZE - 1)
            y1i = jnp.clip(yq.astype(jnp.int32), 0, IMG_SIZE - 1)
            idxv[pl.ds(q, LANES)] = (y1i << 9) + x1i

        copies = []
        for j in range(CHUNK // IDX_PER_STREAM):
            copies.append(pltpu.async_copy(
                patch_hbm.at[idxv.at[pl.ds(j * IDX_PER_STREAM, IDX_PER_STREAM)]],
                rows.at[pl.ds(j * IDX_PER_STREAM, IDX_PER_STREAM)], sem))
        for cp in copies:
            cp.wait()

        lane_iota = lax.iota(jnp.int32, LANES)

        @pl.loop(0, GROUPS)
        def _compute(g):
            q = g * LANES
            xq = xv[pl.ds(q, LANES)]
            yq = yv[pl.ds(q, LANES)]
            x1f = jnp.clip(xq.astype(jnp.int32), 0, IMG_SIZE - 1).astype(jnp.float32)
            y1f = jnp.clip(yq.astype(jnp.int32), 0, IMG_SIZE - 1).astype(jnp.float32)
            wx1 = (x1f + 1.0) - xq
            wx2 = xq - x1f
            wy1 = (y1f + 1.0) - yq
            wy2 = yq - y1f
            cA = wx1 * wy1
            cB = wx2 * wy1
            cC = wx1 * wy2
            cD = wx2 * wy2
            rq = lane_iota + q

            def col(c):
                return plsc.load_gather(
                    rows, [rq, jnp.full((LANES,), c, jnp.int32)])

            fx = cA * col(0) + cB * col(3) + cC * col(6) + cD * col(9)
            fy = cA * col(1) + cB * col(4) + cC * col(7) + cD * col(10)
            fz = cA * col(2) + cB * col(5) + cC * col(8) + cD * col(11)
            n2 = fx * fx + fy * fy + fz * fz
            bits = lax.bitcast_convert_type(n2, jnp.int32)
            seed = jnp.int32(0x5F3759DF) - lax.shift_right_logical(bits, 1)
            r = lax.bitcast_convert_type(seed, jnp.float32)
            h = 0.5 * n2
            r = r * (1.5 - h * r * r)
            r = r * (1.5 - h * r * r)
            r = r * (1.5 - h * r * r)
            plsc.store_scatter(outv, [rq, jnp.full((LANES,), 0, jnp.int32)], fx * r)
            plsc.store_scatter(outv, [rq, jnp.full((LANES,), 1, jnp.int32)], fy * r)
            plsc.store_scatter(outv, [rq, jnp.full((LANES,), 2, jnp.int32)], fz * r)

        pltpu.sync_copy(outv, out_hbm.at[pl.ds(base, CHUNK)])


def kernel(x0, y0, rays):
    n = x0.shape[0]
    n_per_worker = n // NUM_WORKERS
    patch = _build_patch_table(rays)
    mesh = plsc.VectorSubcoreMesh(core_axis_name="c", subcore_axis_name="s")
    run = pl.kernel(
        functools.partial(_sc_body, n_per_worker),
        out_type=jax.ShapeDtypeStruct((n, 3), jnp.float32),
        mesh=mesh,
        scratch_types=[
            pltpu.VMEM((CHUNK,), jnp.float32),
            pltpu.VMEM((CHUNK,), jnp.float32),
            pltpu.VMEM((CHUNK,), jnp.int32),
            pltpu.VMEM((CHUNK, 16), jnp.float32),
            pltpu.VMEM((CHUNK, 3), jnp.float32),
            pltpu.SemaphoreType.DMA,
        ],
        compiler_params=pltpu.CompilerParams(
            needs_layout_passes=False, use_tc_tiling_on_sc=False),
    )
    return run(x0, y0, patch)
